# baseline (speedup 1.0000x reference)
import functools

import jax
import jax.numpy as jnp
from jax.experimental import pallas as pl
from jax.experimental.pallas import tpu as pltpu
from jax.experimental.pallas import tpu_sc as plsc

_W = 128
_BB = 128


def _sc_gather_pairs(pairs):
    tables = [t for t, _ in pairs]
    idxs = [i for _, i in pairs]
    d = tables[0].shape[1]
    mesh = plsc.VectorSubcoreMesh(core_axis_name="core", subcore_axis_name="subcore")
    out_types = tuple(
        jax.ShapeDtypeStruct((idx.size, d), t.dtype) for t, idx in pairs
    )
    np_ = len(pairs)

    @functools.partial(
        pl.kernel,
        out_type=out_types,
        mesh=mesh,
        compiler_params=pltpu.CompilerParams(use_tc_tiling_on_sc=False),
    )
    def k(*refs):
        table_refs = refs[:np_]
        idx_refs = refs[np_: 2 * np_]
        out_refs = refs[2 * np_:]

        for t_hbm, i_hbm, o_hbm in zip(table_refs, idx_refs, out_refs):
            def body(i_vmem, o_vmem, t_hbm=t_hbm):
                pltpu.sync_copy(t_hbm.at[i_vmem.at[0]], o_vmem)

            pltpu.emit_pipeline(
                body,
                grid=(i_hbm.shape[0],),
                in_specs=[pl.BlockSpec((1, _W), index_map=lambda i: (i, 0))],
                out_specs=[pl.BlockSpec((_W, d), index_map=lambda i: (i, 0))],
                core_axis_name=("core", "subcore"),
                dimension_semantics=(pltpu.PARALLEL,),
            )(i_hbm, o_hbm)

    res = k(*tables, *idxs)
    return list(res) if isinstance(res, (tuple, list)) else [res]


def _blockdiag(a):
    d = a.shape[0]
    z = jnp.zeros((d, a.shape[1]), jnp.float32)
    top = jnp.concatenate([a, z], axis=1)
    bot = jnp.concatenate([z, a], axis=1)
    return jnp.concatenate([top, bot], axis=0)


def _pair(row):
    return jnp.concatenate([row, row], axis=1)


def _side_layers(gh0, gh1, gt0, gt1, rels, rel_emb, w1, w2, w3, bmat):
    d = rel_emb.shape[1]
    bb = _BB
    hs = gh0.shape[0] // bb
    w1f = w1[...]
    bmatf = bmat[...]
    w1a = w1f[0:d, :]
    w1b = w1f[d : 2 * d, :]
    r1tab = jnp.dot(rel_emb[...], w1b, preferred_element_type=jnp.float32)
    w1bd = _blockdiag(w1a)
    w2bd = _blockdiag(w2[...])
    w3bd = _blockdiag(jnp.broadcast_to(w3[...], (d, d)))
    r1bd = _blockdiag(r1tab)
    b1p = _pair(bmatf[0:1, :])
    b2p = _pair(bmatf[1:2, :])
    b3 = bmatf[3:4, 0:1]
    nr = rel_emb.shape[0]
    lane_iota = jax.lax.broadcasted_iota(jnp.int32, (1, nr), 1)

    def fold(x3):
        t = jnp.sum(x3, axis=1)
        return t[:, 0:d] + t[:, d : 2 * d]

    def layer(hf, tf, rl2):
        rle = rl2[:, 0:1].astype(jnp.int32)
        rlo = rl2[:, 1:2].astype(jnp.int32)
        ohe = (rle == lane_iota).astype(jnp.float32)
        oho = (rlo == lane_iota).astype(jnp.float32)
        oh2 = jnp.concatenate([ohe, oho], axis=1)
        x = (jnp.dot(hf[...], w1bd, preferred_element_type=jnp.float32)
             + jnp.dot(oh2, r1bd, preferred_element_type=jnp.float32) + b1p)
        x = jnp.maximum(x, 0.0)
        x = jnp.maximum(jnp.dot(x, w2bd, preferred_element_type=jnp.float32) + b2p, 0.0)
        lg = jax.nn.sigmoid(jnp.dot(x, w3bd, preferred_element_type=jnp.float32) + b3)
        p = jnp.exp(lg.reshape(bb, hs, 2 * d))
        tot = jnp.sum(p, axis=1)
        tot = tot[:, 0:d] + tot[:, d : 2 * d]
        p = p / _pair(tot).reshape(bb, 1, 2 * d)
        return fold(p * tf[...].reshape(bb, hs, 2 * d))

    return (layer(gh0, gt0, rels[0]), layer(gh1, gt1, rels[1]))


def _tc_l_body(gh, gt, rels, rel_emb, w1, w2, w3, bmat, out):
    d = rel_emb.shape[1]
    bb = _BB
    hs = gh.shape[0] // bb
    w1f = w1[...]
    bmatf = bmat[...]
    w1a = w1f[0:d, :]
    w1b = w1f[d : 2 * d, :]
    r1tab = jnp.dot(rel_emb[...], w1b, preferred_element_type=jnp.float32)
    w1bd = _blockdiag(w1a)
    w2bd = _blockdiag(w2[...])
    w3bd = _blockdiag(jnp.broadcast_to(w3[...], (d, d)))
    r1bd = _blockdiag(r1tab)
    b1p = _pair(bmatf[0:1, :])
    b2p = _pair(bmatf[1:2, :])
    b3 = bmatf[3:4, 0:1]
    nr = rel_emb.shape[0]
    lane_iota = jax.lax.broadcasted_iota(jnp.int32, (1, nr), 1)

    rl2 = rels[0]
    rle = rl2[:, 0:1].astype(jnp.int32)
    rlo = rl2[:, 1:2].astype(jnp.int32)
    ohe = (rle == lane_iota).astype(jnp.float32)
    oho = (rlo == lane_iota).astype(jnp.float32)
    oh2 = jnp.concatenate([ohe, oho], axis=1)
    x = (jnp.dot(gh[...], w1bd, preferred_element_type=jnp.float32)
         + jnp.dot(oh2, r1bd, preferred_element_type=jnp.float32) + b1p)
    x = jnp.maximum(x, 0.0)
    x = jnp.maximum(jnp.dot(x, w2bd, preferred_element_type=jnp.float32) + b2p, 0.0)
    lg = jax.nn.sigmoid(jnp.dot(x, w3bd, preferred_element_type=jnp.float32) + b3)
    p = jnp.exp(lg.reshape(bb, hs, 2 * d))
    tot = jnp.sum(p, axis=1)
    tot = tot[:, 0:d] + tot[:, d : 2 * d]
    p = p / _pair(tot).reshape(bb, 1, 2 * d)
    t = jnp.sum(p * gt[...].reshape(bb, hs, 2 * d), axis=1)
    out[...] = t[:, 0:d] + t[:, d : 2 * d]


def _side_embedding(gh0, gh1, gt0, gt1, g0, rels,
                    rel_emb, w1, w2, w3, wagg, bmat):
    d = rel_emb.shape[1]
    bb = _BB
    hs = gh0.shape[0] // bb
    w1f = w1[...]
    waggf = wagg[...]
    bmatf = bmat[...]
    w1a = w1f[0:d, :]
    w1b = w1f[d : 2 * d, :]
    r1tab = jnp.dot(rel_emb[...], w1b, preferred_element_type=jnp.float32)
    w1bd = _blockdiag(w1a)
    w2bd = _blockdiag(w2[...])
    w3bd = _blockdiag(jnp.broadcast_to(w3[...], (d, d)))
    r1bd = _blockdiag(r1tab)
    b1p = _pair(bmatf[0:1, :])
    b2p = _pair(bmatf[1:2, :])
    bagg = bmatf[2:3, :]
    b3 = bmatf[3:4, 0:1]
    nr = rel_emb.shape[0]
    lane_iota = jax.lax.broadcasted_iota(jnp.int32, (1, nr), 1)

    def fold(x3):
        t = jnp.sum(x3, axis=1)
        return t[:, 0:d] + t[:, d : 2 * d]

    def layer(hf, tf, rl2):
        rle = rl2[:, 0:1].astype(jnp.int32)
        rlo = rl2[:, 1:2].astype(jnp.int32)
        ohe = (rle == lane_iota).astype(jnp.float32)
        oho = (rlo == lane_iota).astype(jnp.float32)
        oh2 = jnp.concatenate([ohe, oho], axis=1)
        x = (jnp.dot(hf[...], w1bd, preferred_element_type=jnp.float32)
             + jnp.dot(oh2, r1bd, preferred_element_type=jnp.float32) + b1p)
        x = jnp.maximum(x, 0.0)
        x = jnp.maximum(jnp.dot(x, w2bd, preferred_element_type=jnp.float32) + b2p, 0.0)
        lg = jax.nn.sigmoid(jnp.dot(x, w3bd, preferred_element_type=jnp.float32) + b3)
        p = jnp.exp(lg.reshape(bb, hs, 2 * d))
        tot = jnp.sum(p, axis=1)
        tot = tot[:, 0:d] + tot[:, d : 2 * d]
        p = p / _pair(tot).reshape(bb, 1, 2 * d)
        return fold(p * tf[...].reshape(bb, hs, 2 * d))

    e0 = fold(g0[...].reshape(bb, hs, 2 * d)) * (1.0 / (2 * hs))
    e1 = layer(gh0, gt0, rels[0])
    e2 = layer(gh1, gt1, rels[1])
    y = (jnp.dot(e0, waggf[0:d, :], preferred_element_type=jnp.float32)
         + jnp.dot(e1, waggf[d : 2 * d, :], preferred_element_type=jnp.float32)
         + jnp.dot(e2, waggf[2 * d : 3 * d, :], preferred_element_type=jnp.float32)
         + bagg)
    return jax.nn.sigmoid(y)


def _tc_u_body(gh0, gh1, gt0, gt1, g0, rels, rel_emb, w1, w2, w3, wagg, bmat,
               out):
    out[...] = _side_embedding(gh0, gh1, gt0, gt1, g0, rels,
                               rel_emb, w1, w2, w3, wagg, bmat)


def _tc_vl_body(gh0, gh1, gt0, gt1, rels, rel_emb, w1, w2, w3, bmat, out):
    e1, e2 = _side_layers(gh0, gh1, gt0, gt1, rels, rel_emb, w1, w2, w3, bmat)
    out[...] = jnp.concatenate([e1, e2], axis=1)


def _tc_vf_body(g0, e1, e2, eu, wagg, bmat, out):
    d = eu.shape[-1]
    bb = _BB
    hs = g0.shape[0] // bb
    waggf = wagg[...]
    bmatf = bmat[...]
    x3 = g0[...].reshape(bb, hs, 2 * d)
    t = jnp.sum(x3, axis=1)
    e0 = (t[:, 0:d] + t[:, d : 2 * d]) * (1.0 / (2 * hs))
    y = (jnp.dot(e0, waggf[0:d, :], preferred_element_type=jnp.float32)
         + jnp.dot(e1[...], waggf[d : 2 * d, :], preferred_element_type=jnp.float32)
         + jnp.dot(e2[...], waggf[2 * d : 3 * d, :], preferred_element_type=jnp.float32)
         + bmatf[2:3, :])
    ev = jax.nn.sigmoid(y)
    out[...] = jax.nn.sigmoid(jnp.sum(eu[...] * ev, axis=1, keepdims=True))


def _tc_side_specs(b, s, d, nr, with_eu):
    bb = _BB
    grid = (b // bb,)
    hbs = bb * s // 2
    nlb = b * s // 2 // hbs
    in_specs = [
        pl.BlockSpec((hbs, 2 * d), lambda i: (i, 0)),
        pl.BlockSpec((hbs, 2 * d), lambda i: (i + nlb, 0)),
        pl.BlockSpec((hbs, 2 * d), lambda i: (i, 0)),
        pl.BlockSpec((hbs, 2 * d), lambda i: (i + nlb, 0)),
        pl.BlockSpec((hbs, 2 * d), lambda i: (i, 0)),
        pl.BlockSpec((2, hbs, 2), lambda i: (0, i, 0)),
        pl.BlockSpec((nr, d), lambda i: (0, 0)),
        pl.BlockSpec((2 * d, d), lambda i: (0, 0)),
        pl.BlockSpec((d, d), lambda i: (0, 0)),
        pl.BlockSpec((d, 1), lambda i: (0, 0)),
        pl.BlockSpec((3 * d, d), lambda i: (0, 0)),
        pl.BlockSpec((8, d), lambda i: (0, 0)),
    ]
    if with_eu:
        in_specs.append(pl.BlockSpec((bb, d), lambda i: (i, 0)))
        out_specs = pl.BlockSpec((bb, 1), lambda i: (i, 0))
        out_shape = jax.ShapeDtypeStruct((b, 1), jnp.float32)
    else:
        out_specs = pl.BlockSpec((bb, d), lambda i: (i, 0))
        out_shape = jax.ShapeDtypeStruct((b, d), jnp.float32)
    return grid, in_specs, out_specs, out_shape


def kernel(u_entities, u_heads, u_relations, u_tails,
           v_entities, v_heads, v_relations, v_tails,
           entity_emb, rec_emb, rel_emb,
           W1, b1, W2, b2, W3, b3, Wagg, bagg):
    b, s = u_entities.shape
    d = entity_emb.shape[1]
    nr = rel_emb.shape[0]
    ls = u_heads.shape[0]
    n = b * s

    g_uh, g_ut = _sc_gather_pairs([
        (entity_emb, u_heads.reshape(-1, _W)),
        (entity_emb, u_tails.reshape(-1, _W)),
    ])
    (g_ue,) = _sc_gather_pairs([(rec_emb, u_entities.reshape(-1, _W))])
    vhw = v_heads.reshape(-1, _W)
    vtw = v_tails.reshape(-1, _W)
    hw = vhw.shape[0] // 2
    g_vh0, g_vt0 = _sc_gather_pairs([
        (entity_emb, vhw[0:hw]),
        (entity_emb, vtw[0:hw]),
    ])
    (g_ve,) = _sc_gather_pairs([(entity_emb, v_entities.reshape(-1, _W))])
    g_vh1, g_vt1 = _sc_gather_pairs([
        (entity_emb, vhw[hw:]),
        (entity_emb, vtw[hw:]),
    ])
    pk = lambda g: g.reshape(-1, 2 * d)

    urels = u_relations.astype(jnp.int8).reshape(ls, n // 2, 2)
    vrels = v_relations.astype(jnp.int8).reshape(ls, n // 2, 2)
    bmat = jnp.concatenate([
        b1[None, :], b2[None, :], bagg[None, :],
        jnp.broadcast_to(b3, (d,))[None, :], jnp.zeros((4, d), jnp.float32),
    ], axis=0)
    weights = (rel_emb, W1, W2, W3, Wagg, bmat)

    grid, in_specs_u, out_specs_u, out_shape_u = _tc_side_specs(b, s, d, nr, False)
    eu = pl.pallas_call(
        _tc_u_body, grid=grid, in_specs=in_specs_u,
        out_specs=out_specs_u, out_shape=out_shape_u,
    )(pk(g_uh), pk(g_uh), pk(g_ut), pk(g_ut), pk(g_ue), urels, *weights)

    bb = _BB
    hbs = bb * s // 2

    def _layer_call(gh, gt, l):
        return pl.pallas_call(
            _tc_l_body, grid=(b // bb,),
            in_specs=[
                pl.BlockSpec((hbs, 2 * d), lambda i: (i, 0)),
                pl.BlockSpec((hbs, 2 * d), lambda i: (i, 0)),
                pl.BlockSpec((1, hbs, 2), lambda i, l=l: (l, i, 0)),
                pl.BlockSpec((nr, d), lambda i: (0, 0)),
                pl.BlockSpec((2 * d, d), lambda i: (0, 0)),
                pl.BlockSpec((d, d), lambda i: (0, 0)),
                pl.BlockSpec((d, 1), lambda i: (0, 0)),
                pl.BlockSpec((8, d), lambda i: (0, 0)),
            ],
            out_specs=pl.BlockSpec((bb, d), lambda i: (i, 0)),
            out_shape=jax.ShapeDtypeStruct((b, d), jnp.float32),
        )(gh, gt, vrels, rel_emb, W1, W2, W3, bmat)

    e1v = _layer_call(pk(g_vh0), pk(g_vt0), 0)
    e2v = _layer_call(pk(g_vh1), pk(g_vt1), 1)
    out = pl.pallas_call(
        _tc_vf_body, grid=(b // bb,),
        in_specs=[
            pl.BlockSpec((hbs, 2 * d), lambda i: (i, 0)),
            pl.BlockSpec((bb, d), lambda i: (i, 0)),
            pl.BlockSpec((bb, d), lambda i: (i, 0)),
            pl.BlockSpec((bb, d), lambda i: (i, 0)),
            pl.BlockSpec((3 * d, d), lambda i: (0, 0)),
            pl.BlockSpec((8, d), lambda i: (0, 0)),
        ],
        out_specs=pl.BlockSpec((bb, 1), lambda i: (i, 0)),
        out_shape=jax.ShapeDtypeStruct((b, 1), jnp.float32),
    )(pk(g_ve), e1v, e2v, eu, Wagg, bmat)
    return out.reshape(-1)


# speedup vs baseline: 1.0141x; 1.0141x over previous
import functools

import jax
import jax.numpy as jnp
from jax.experimental import pallas as pl
from jax.experimental.pallas import tpu as pltpu
from jax.experimental.pallas import tpu_sc as plsc

_W = 128
_BB = 128


def _sc_gather_pairs(pairs):
    tables = [t for t, _ in pairs]
    idxs = [i for _, i in pairs]
    d = tables[0].shape[1]
    mesh = plsc.VectorSubcoreMesh(core_axis_name="core", subcore_axis_name="subcore")
    out_types = tuple(
        jax.ShapeDtypeStruct((idx.size, d), t.dtype) for t, idx in pairs
    )
    np_ = len(pairs)

    @functools.partial(
        pl.kernel,
        out_type=out_types,
        mesh=mesh,
        compiler_params=pltpu.CompilerParams(use_tc_tiling_on_sc=False),
    )
    def k(*refs):
        table_refs = refs[:np_]
        idx_refs = refs[np_: 2 * np_]
        out_refs = refs[2 * np_:]

        for t_hbm, i_hbm, o_hbm in zip(table_refs, idx_refs, out_refs):
            def body(i_vmem, o_vmem, t_hbm=t_hbm):
                pltpu.sync_copy(t_hbm.at[i_vmem.at[0]], o_vmem)

            pltpu.emit_pipeline(
                body,
                grid=(i_hbm.shape[0],),
                in_specs=[pl.BlockSpec((1, _W), index_map=lambda i: (i, 0))],
                out_specs=[pl.BlockSpec((_W, d), index_map=lambda i: (i, 0))],
                core_axis_name=("core", "subcore"),
                dimension_semantics=(pltpu.PARALLEL,),
            )(i_hbm, o_hbm)

    res = k(*tables, *idxs)
    return list(res) if isinstance(res, (tuple, list)) else [res]


def _blockdiag(a):
    d = a.shape[0]
    z = jnp.zeros((d, a.shape[1]), jnp.float32)
    top = jnp.concatenate([a, z], axis=1)
    bot = jnp.concatenate([z, a], axis=1)
    return jnp.concatenate([top, bot], axis=0)


def _pair(row):
    return jnp.concatenate([row, row], axis=1)


def _side_layers(gh0, gh1, gt0, gt1, rels, rel_emb, w1, w2, w3, bmat):
    d = rel_emb.shape[1]
    bb = _BB
    hs = gh0.shape[0] // bb
    w1f = w1[...]
    bmatf = bmat[...]
    w1a = w1f[0:d, :]
    w1b = w1f[d : 2 * d, :]
    r1tab = jnp.dot(rel_emb[...], w1b, preferred_element_type=jnp.float32)
    w1bd = _blockdiag(w1a)
    w2bd = _blockdiag(w2[...])
    w3bd = _blockdiag(jnp.broadcast_to(w3[...], (d, d)))
    r1bd = _blockdiag(r1tab)
    b1p = _pair(bmatf[0:1, :])
    b2p = _pair(bmatf[1:2, :])
    b3 = bmatf[3:4, 0:1]
    nr = rel_emb.shape[0]
    lane_iota = jax.lax.broadcasted_iota(jnp.int32, (1, nr), 1)

    def fold(x3):
        t = jnp.sum(x3, axis=1)
        return t[:, 0:d] + t[:, d : 2 * d]

    def layer(hf, tf, rl2):
        rle = rl2[:, 0:1].astype(jnp.int32)
        rlo = rl2[:, 1:2].astype(jnp.int32)
        ohe = (rle == lane_iota).astype(jnp.float32)
        oho = (rlo == lane_iota).astype(jnp.float32)
        oh2 = jnp.concatenate([ohe, oho], axis=1)
        x = (jnp.dot(hf[...], w1bd, preferred_element_type=jnp.float32)
             + jnp.dot(oh2, r1bd, preferred_element_type=jnp.float32) + b1p)
        x = jnp.maximum(x, 0.0)
        x = jnp.maximum(jnp.dot(x, w2bd, preferred_element_type=jnp.float32) + b2p, 0.0)
        lg = jax.nn.sigmoid(jnp.dot(x, w3bd, preferred_element_type=jnp.float32) + b3)
        p = jnp.exp(lg.reshape(bb, hs, 2 * d))
        tot = jnp.sum(p, axis=1)
        tot = tot[:, 0:d] + tot[:, d : 2 * d]
        p = p / _pair(tot).reshape(bb, 1, 2 * d)
        return fold(p * tf[...].reshape(bb, hs, 2 * d))

    return (layer(gh0, gt0, rels[0]), layer(gh1, gt1, rels[1]))


def _tc_l_body(gh, gt, rels, rel_emb, w1, w2, w3, bmat, out):
    d = rel_emb.shape[1]
    bb = _BB
    hs = gh.shape[0] // bb
    w1f = w1[...]
    bmatf = bmat[...]
    w1a = w1f[0:d, :]
    w1b = w1f[d : 2 * d, :]
    r1tab = jnp.dot(rel_emb[...], w1b, preferred_element_type=jnp.float32)
    w1bd = _blockdiag(w1a)
    w2bd = _blockdiag(w2[...])
    w3bd = _blockdiag(jnp.broadcast_to(w3[...], (d, d)))
    r1bd = _blockdiag(r1tab)
    b1p = _pair(bmatf[0:1, :])
    b2p = _pair(bmatf[1:2, :])
    b3 = bmatf[3:4, 0:1]
    nr = rel_emb.shape[0]
    lane_iota = jax.lax.broadcasted_iota(jnp.int32, (1, nr), 1)

    rl2 = rels[0]
    rle = rl2[:, 0:1].astype(jnp.int32)
    rlo = rl2[:, 1:2].astype(jnp.int32)
    ohe = (rle == lane_iota).astype(jnp.float32)
    oho = (rlo == lane_iota).astype(jnp.float32)
    oh2 = jnp.concatenate([ohe, oho], axis=1)
    x = (jnp.dot(gh[...], w1bd, preferred_element_type=jnp.float32)
         + jnp.dot(oh2, r1bd, preferred_element_type=jnp.float32) + b1p)
    x = jnp.maximum(x, 0.0)
    x = jnp.maximum(jnp.dot(x, w2bd, preferred_element_type=jnp.float32) + b2p, 0.0)
    lg = jax.nn.sigmoid(jnp.dot(x, w3bd, preferred_element_type=jnp.float32) + b3)
    p = jnp.exp(lg.reshape(bb, hs, 2 * d))
    tot = jnp.sum(p, axis=1)
    tot = tot[:, 0:d] + tot[:, d : 2 * d]
    p = p / _pair(tot).reshape(bb, 1, 2 * d)
    t = jnp.sum(p * gt[...].reshape(bb, hs, 2 * d), axis=1)
    out[...] = t[:, 0:d] + t[:, d : 2 * d]


def _side_embedding(gh0, gh1, gt0, gt1, g0, rels,
                    rel_emb, w1, w2, w3, wagg, bmat):
    d = rel_emb.shape[1]
    bb = _BB
    hs = gh0.shape[0] // bb
    w1f = w1[...]
    waggf = wagg[...]
    bmatf = bmat[...]
    w1a = w1f[0:d, :]
    w1b = w1f[d : 2 * d, :]
    r1tab = jnp.dot(rel_emb[...], w1b, preferred_element_type=jnp.float32)
    w1bd = _blockdiag(w1a)
    w2bd = _blockdiag(w2[...])
    w3bd = _blockdiag(jnp.broadcast_to(w3[...], (d, d)))
    r1bd = _blockdiag(r1tab)
    b1p = _pair(bmatf[0:1, :])
    b2p = _pair(bmatf[1:2, :])
    bagg = bmatf[2:3, :]
    b3 = bmatf[3:4, 0:1]
    nr = rel_emb.shape[0]
    lane_iota = jax.lax.broadcasted_iota(jnp.int32, (1, nr), 1)

    def fold(x3):
        t = jnp.sum(x3, axis=1)
        return t[:, 0:d] + t[:, d : 2 * d]

    def layer(hf, tf, rl2):
        rle = rl2[:, 0:1].astype(jnp.int32)
        rlo = rl2[:, 1:2].astype(jnp.int32)
        ohe = (rle == lane_iota).astype(jnp.float32)
        oho = (rlo == lane_iota).astype(jnp.float32)
        oh2 = jnp.concatenate([ohe, oho], axis=1)
        x = (jnp.dot(hf[...], w1bd, preferred_element_type=jnp.float32)
             + jnp.dot(oh2, r1bd, preferred_element_type=jnp.float32) + b1p)
        x = jnp.maximum(x, 0.0)
        x = jnp.maximum(jnp.dot(x, w2bd, preferred_element_type=jnp.float32) + b2p, 0.0)
        lg = jax.nn.sigmoid(jnp.dot(x, w3bd, preferred_element_type=jnp.float32) + b3)
        p = jnp.exp(lg.reshape(bb, hs, 2 * d))
        tot = jnp.sum(p, axis=1)
        tot = tot[:, 0:d] + tot[:, d : 2 * d]
        p = p / _pair(tot).reshape(bb, 1, 2 * d)
        return fold(p * tf[...].reshape(bb, hs, 2 * d))

    e0 = fold(g0[...].reshape(bb, hs, 2 * d)) * (1.0 / (2 * hs))
    e1 = layer(gh0, gt0, rels[0])
    e2 = layer(gh1, gt1, rels[1])
    y = (jnp.dot(e0, waggf[0:d, :], preferred_element_type=jnp.float32)
         + jnp.dot(e1, waggf[d : 2 * d, :], preferred_element_type=jnp.float32)
         + jnp.dot(e2, waggf[2 * d : 3 * d, :], preferred_element_type=jnp.float32)
         + bagg)
    return jax.nn.sigmoid(y)


def _tc_u_body(gh0, gh1, gt0, gt1, g0, rels, rel_emb, w1, w2, w3, wagg, bmat,
               out):
    out[...] = _side_embedding(gh0, gh1, gt0, gt1, g0, rels,
                               rel_emb, w1, w2, w3, wagg, bmat)


def _tc_vl_body(gh0, gh1, gt0, gt1, rels, rel_emb, w1, w2, w3, bmat, out):
    e1, e2 = _side_layers(gh0, gh1, gt0, gt1, rels, rel_emb, w1, w2, w3, bmat)
    out[...] = jnp.concatenate([e1, e2], axis=1)


def _tc_vf_body(g0, evl, eu, wagg, bmat, out):
    d = eu.shape[-1]
    bb = _BB
    hs = g0.shape[0] // bb
    waggf = wagg[...]
    bmatf = bmat[...]
    x3 = g0[...].reshape(bb, hs, 2 * d)
    t = jnp.sum(x3, axis=1)
    e0 = (t[:, 0:d] + t[:, d : 2 * d]) * (1.0 / (2 * hs))
    evlf = evl[...]
    y = (jnp.dot(e0, waggf[0:d, :], preferred_element_type=jnp.float32)
         + jnp.dot(evlf[:, 0:d], waggf[d : 2 * d, :], preferred_element_type=jnp.float32)
         + jnp.dot(evlf[:, d : 2 * d], waggf[2 * d : 3 * d, :], preferred_element_type=jnp.float32)
         + bmatf[2:3, :])
    ev = jax.nn.sigmoid(y)
    out[...] = jax.nn.sigmoid(jnp.sum(eu[...] * ev, axis=1, keepdims=True))


def _tc_side_specs(b, s, d, nr, with_eu):
    bb = _BB
    grid = (b // bb,)
    hbs = bb * s // 2
    nlb = b * s // 2 // hbs
    in_specs = [
        pl.BlockSpec((hbs, 2 * d), lambda i: (i, 0)),
        pl.BlockSpec((hbs, 2 * d), lambda i: (i + nlb, 0)),
        pl.BlockSpec((hbs, 2 * d), lambda i: (i, 0)),
        pl.BlockSpec((hbs, 2 * d), lambda i: (i + nlb, 0)),
        pl.BlockSpec((hbs, 2 * d), lambda i: (i, 0)),
        pl.BlockSpec((2, hbs, 2), lambda i: (0, i, 0)),
        pl.BlockSpec((nr, d), lambda i: (0, 0)),
        pl.BlockSpec((2 * d, d), lambda i: (0, 0)),
        pl.BlockSpec((d, d), lambda i: (0, 0)),
        pl.BlockSpec((d, 1), lambda i: (0, 0)),
        pl.BlockSpec((3 * d, d), lambda i: (0, 0)),
        pl.BlockSpec((8, d), lambda i: (0, 0)),
    ]
    if with_eu:
        in_specs.append(pl.BlockSpec((bb, d), lambda i: (i, 0)))
        out_specs = pl.BlockSpec((bb, 1), lambda i: (i, 0))
        out_shape = jax.ShapeDtypeStruct((b, 1), jnp.float32)
    else:
        out_specs = pl.BlockSpec((bb, d), lambda i: (i, 0))
        out_shape = jax.ShapeDtypeStruct((b, d), jnp.float32)
    return grid, in_specs, out_specs, out_shape


def kernel(u_entities, u_heads, u_relations, u_tails,
           v_entities, v_heads, v_relations, v_tails,
           entity_emb, rec_emb, rel_emb,
           W1, b1, W2, b2, W3, b3, Wagg, bagg):
    b, s = u_entities.shape
    d = entity_emb.shape[1]
    nr = rel_emb.shape[0]
    ls = u_heads.shape[0]
    n = b * s

    g_uh, g_ut = _sc_gather_pairs([
        (entity_emb, u_heads.reshape(-1, _W)),
        (entity_emb, u_tails.reshape(-1, _W)),
    ])
    (g_ue,) = _sc_gather_pairs([(rec_emb, u_entities.reshape(-1, _W))])
    g_vh, g_vt = _sc_gather_pairs([
        (entity_emb, v_heads.reshape(-1, _W)),
        (entity_emb, v_tails.reshape(-1, _W)),
    ])
    (g_ve,) = _sc_gather_pairs([(entity_emb, v_entities.reshape(-1, _W))])
    pk = lambda g: g.reshape(-1, 2 * d)

    urels = u_relations.astype(jnp.int8).reshape(ls, n // 2, 2)
    vrels = v_relations.astype(jnp.int8).reshape(ls, n // 2, 2)
    bmat = jnp.concatenate([
        b1[None, :], b2[None, :], bagg[None, :],
        jnp.broadcast_to(b3, (d,))[None, :], jnp.zeros((4, d), jnp.float32),
    ], axis=0)
    weights = (rel_emb, W1, W2, W3, Wagg, bmat)

    grid, in_specs_u, out_specs_u, out_shape_u = _tc_side_specs(b, s, d, nr, False)
    eu = pl.pallas_call(
        _tc_u_body, grid=grid, in_specs=in_specs_u,
        out_specs=out_specs_u, out_shape=out_shape_u,
    )(pk(g_uh), pk(g_uh), pk(g_ut), pk(g_ut), pk(g_ue), urels, *weights)

    bb = _BB
    hbs = bb * s // 2
    nlb = b * s // 2 // hbs
    evl = pl.pallas_call(
        _tc_vl_body, grid=(b // bb,),
        in_specs=[
            pl.BlockSpec((hbs, 2 * d), lambda i: (i, 0)),
            pl.BlockSpec((hbs, 2 * d), lambda i: (i + nlb, 0)),
            pl.BlockSpec((hbs, 2 * d), lambda i: (i, 0)),
            pl.BlockSpec((hbs, 2 * d), lambda i: (i + nlb, 0)),
            pl.BlockSpec((2, hbs, 2), lambda i: (0, i, 0)),
            pl.BlockSpec((nr, d), lambda i: (0, 0)),
            pl.BlockSpec((2 * d, d), lambda i: (0, 0)),
            pl.BlockSpec((d, d), lambda i: (0, 0)),
            pl.BlockSpec((d, 1), lambda i: (0, 0)),
            pl.BlockSpec((8, d), lambda i: (0, 0)),
        ],
        out_specs=pl.BlockSpec((bb, 2 * d), lambda i: (i, 0)),
        out_shape=jax.ShapeDtypeStruct((b, 2 * d), jnp.float32),
    )(pk(g_vh), pk(g_vh), pk(g_vt), pk(g_vt), vrels,
      rel_emb, W1, W2, W3, bmat)
    out = pl.pallas_call(
        _tc_vf_body, grid=(b // bb,),
        in_specs=[
            pl.BlockSpec((hbs, 2 * d), lambda i: (i, 0)),
            pl.BlockSpec((bb, 2 * d), lambda i: (i, 0)),
            pl.BlockSpec((bb, d), lambda i: (i, 0)),
            pl.BlockSpec((3 * d, d), lambda i: (0, 0)),
            pl.BlockSpec((8, d), lambda i: (0, 0)),
        ],
        out_specs=pl.BlockSpec((bb, 1), lambda i: (i, 0)),
        out_shape=jax.ShapeDtypeStruct((b, 1), jnp.float32),
    )(pk(g_ve), evl, eu, Wagg, bmat)
    return out.reshape(-1)


# speedup vs baseline: 1.0144x; 1.0003x over previous
import functools

import jax
import jax.numpy as jnp
from jax.experimental import pallas as pl
from jax.experimental.pallas import tpu as pltpu
from jax.experimental.pallas import tpu_sc as plsc

_W = 128
_BB = 128


def _sc_gather_pairs(pairs):
    tables = [t for t, _ in pairs]
    idxs = [i for _, i in pairs]
    d = tables[0].shape[1]
    mesh = plsc.VectorSubcoreMesh(core_axis_name="core", subcore_axis_name="subcore")
    out_types = tuple(
        jax.ShapeDtypeStruct((idx.size, d), t.dtype) for t, idx in pairs
    )
    np_ = len(pairs)

    @functools.partial(
        pl.kernel,
        out_type=out_types,
        mesh=mesh,
        compiler_params=pltpu.CompilerParams(use_tc_tiling_on_sc=False),
    )
    def k(*refs):
        table_refs = refs[:np_]
        idx_refs = refs[np_: 2 * np_]
        out_refs = refs[2 * np_:]

        for t_hbm, i_hbm, o_hbm in zip(table_refs, idx_refs, out_refs):
            def body(i_vmem, o_vmem, t_hbm=t_hbm):
                pltpu.sync_copy(t_hbm.at[i_vmem.at[0]], o_vmem)

            pltpu.emit_pipeline(
                body,
                grid=(i_hbm.shape[0],),
                in_specs=[pl.BlockSpec((1, _W), index_map=lambda i: (i, 0))],
                out_specs=[pl.BlockSpec((_W, d), index_map=lambda i: (i, 0))],
                core_axis_name=("core", "subcore"),
                dimension_semantics=(pltpu.PARALLEL,),
            )(i_hbm, o_hbm)

    res = k(*tables, *idxs)
    return list(res) if isinstance(res, (tuple, list)) else [res]


def _blockdiag(a):
    d = a.shape[0]
    z = jnp.zeros((d, a.shape[1]), jnp.float32)
    top = jnp.concatenate([a, z], axis=1)
    bot = jnp.concatenate([z, a], axis=1)
    return jnp.concatenate([top, bot], axis=0)


def _pair(row):
    return jnp.concatenate([row, row], axis=1)


def _side_layers(gh0, gh1, gt0, gt1, rels, rel_emb, w1, w2, w3, bmat):
    d = rel_emb.shape[1]
    bb = _BB
    hs = gh0.shape[0] // bb
    w1f = w1[...]
    bmatf = bmat[...]
    w1a = w1f[0:d, :]
    w1b = w1f[d : 2 * d, :]
    r1tab = jnp.dot(rel_emb[...], w1b, preferred_element_type=jnp.float32)
    w1bd = _blockdiag(w1a)
    w2bd = _blockdiag(w2[...])
    w3bd = _blockdiag(jnp.broadcast_to(w3[...], (d, d)))
    r1bd = _blockdiag(r1tab)
    b1p = _pair(bmatf[0:1, :])
    b2p = _pair(bmatf[1:2, :])
    b3 = bmatf[3:4, 0:1]
    nr = rel_emb.shape[0]
    lane_iota = jax.lax.broadcasted_iota(jnp.int32, (1, nr), 1)

    def fold(x3):
        t = jnp.sum(x3, axis=1)
        return t[:, 0:d] + t[:, d : 2 * d]

    def layer(hf, tf, rl2):
        rle = rl2[:, 0:1].astype(jnp.int32)
        rlo = rl2[:, 1:2].astype(jnp.int32)
        ohe = (rle == lane_iota).astype(jnp.float32)
        oho = (rlo == lane_iota).astype(jnp.float32)
        oh2 = jnp.concatenate([ohe, oho], axis=1)
        x = (jnp.dot(hf[...], w1bd, preferred_element_type=jnp.float32)
             + jnp.dot(oh2, r1bd, preferred_element_type=jnp.float32) + b1p)
        x = jnp.maximum(x, 0.0)
        x = jnp.maximum(jnp.dot(x, w2bd, preferred_element_type=jnp.float32) + b2p, 0.0)
        lg = jax.nn.sigmoid(jnp.dot(x, w3bd, preferred_element_type=jnp.float32) + b3)
        p = jnp.exp(lg.reshape(bb, hs, 2 * d))
        tot = jnp.sum(p, axis=1)
        tot = tot[:, 0:d] + tot[:, d : 2 * d]
        p = p / _pair(tot).reshape(bb, 1, 2 * d)
        return fold(p * tf[...].reshape(bb, hs, 2 * d))

    return (layer(gh0, gt0, rels[0]), layer(gh1, gt1, rels[1]))


def _side_embedding(gh0, gh1, gt0, gt1, g0, rels,
                    rel_emb, w1, w2, w3, wagg, bmat):
    d = rel_emb.shape[1]
    bb = _BB
    hs = gh0.shape[0] // bb
    w1f = w1[...]
    waggf = wagg[...]
    bmatf = bmat[...]
    w1a = w1f[0:d, :]
    w1b = w1f[d : 2 * d, :]
    r1tab = jnp.dot(rel_emb[...], w1b, preferred_element_type=jnp.float32)
    w1bd = _blockdiag(w1a)
    w2bd = _blockdiag(w2[...])
    w3bd = _blockdiag(jnp.broadcast_to(w3[...], (d, d)))
    r1bd = _blockdiag(r1tab)
    b1p = _pair(bmatf[0:1, :])
    b2p = _pair(bmatf[1:2, :])
    bagg = bmatf[2:3, :]
    b3 = bmatf[3:4, 0:1]
    nr = rel_emb.shape[0]
    lane_iota = jax.lax.broadcasted_iota(jnp.int32, (1, nr), 1)

    def fold(x3):
        t = jnp.sum(x3, axis=1)
        return t[:, 0:d] + t[:, d : 2 * d]

    def layer(hf, tf, rl2):
        rle = rl2[:, 0:1].astype(jnp.int32)
        rlo = rl2[:, 1:2].astype(jnp.int32)
        ohe = (rle == lane_iota).astype(jnp.float32)
        oho = (rlo == lane_iota).astype(jnp.float32)
        oh2 = jnp.concatenate([ohe, oho], axis=1)
        x = (jnp.dot(hf[...], w1bd, preferred_element_type=jnp.float32)
             + jnp.dot(oh2, r1bd, preferred_element_type=jnp.float32) + b1p)
        x = jnp.maximum(x, 0.0)
        x = jnp.maximum(jnp.dot(x, w2bd, preferred_element_type=jnp.float32) + b2p, 0.0)
        lg = jax.nn.sigmoid(jnp.dot(x, w3bd, preferred_element_type=jnp.float32) + b3)
        p = jnp.exp(lg.reshape(bb, hs, 2 * d))
        tot = jnp.sum(p, axis=1)
        tot = tot[:, 0:d] + tot[:, d : 2 * d]
        p = p / _pair(tot).reshape(bb, 1, 2 * d)
        return fold(p * tf[...].reshape(bb, hs, 2 * d))

    e0 = fold(g0[...].reshape(bb, hs, 2 * d)) * (1.0 / (2 * hs))
    e1 = layer(gh0, gt0, rels[0])
    e2 = layer(gh1, gt1, rels[1])
    y = (jnp.dot(e0, waggf[0:d, :], preferred_element_type=jnp.float32)
         + jnp.dot(e1, waggf[d : 2 * d, :], preferred_element_type=jnp.float32)
         + jnp.dot(e2, waggf[2 * d : 3 * d, :], preferred_element_type=jnp.float32)
         + bagg)
    return jax.nn.sigmoid(y)


def _tc_u_body(gh0, gh1, gt0, gt1, g0, rels, rel_emb, w1, w2, w3, wagg, bmat,
               out):
    out[...] = _side_embedding(gh0, gh1, gt0, gt1, g0, rels,
                               rel_emb, w1, w2, w3, wagg, bmat)


def _tc_vl_body(gh0, gh1, gt0, gt1, rels, rel_emb, w1, w2, w3, bmat, out):
    e1, e2 = _side_layers(gh0, gh1, gt0, gt1, rels, rel_emb, w1, w2, w3, bmat)
    out[...] = jnp.concatenate([e1, e2], axis=1)


def _tc_vf_body(g0, evl, eu, wagg, bmat, out):
    d = eu.shape[-1]
    bb = _BB
    hs = g0.shape[0] // bb
    waggf = wagg[...]
    bmatf = bmat[...]
    x3 = g0[...].reshape(bb, hs, 2 * d)
    t = jnp.sum(x3, axis=1)
    e0 = (t[:, 0:d] + t[:, d : 2 * d]) * (1.0 / (2 * hs))
    evlf = evl[...]
    y = (jnp.dot(e0, waggf[0:d, :], preferred_element_type=jnp.float32)
         + jnp.dot(evlf[:, 0:d], waggf[d : 2 * d, :], preferred_element_type=jnp.float32)
         + jnp.dot(evlf[:, d : 2 * d], waggf[2 * d : 3 * d, :], preferred_element_type=jnp.float32)
         + bmatf[2:3, :])
    ev = jax.nn.sigmoid(y)
    out[...] = jax.nn.sigmoid(jnp.sum(eu[...] * ev, axis=1, keepdims=True))


def _tc_side_specs(b, s, d, nr, with_eu):
    bb = _BB
    grid = (b // bb,)
    hbs = bb * s // 2
    nlb = b * s // 2 // hbs
    in_specs = [
        pl.BlockSpec((hbs, 2 * d), lambda i: (i, 0)),
        pl.BlockSpec((hbs, 2 * d), lambda i: (i + nlb, 0)),
        pl.BlockSpec((hbs, 2 * d), lambda i: (i, 0)),
        pl.BlockSpec((hbs, 2 * d), lambda i: (i + nlb, 0)),
        pl.BlockSpec((hbs, 2 * d), lambda i: (i, 0)),
        pl.BlockSpec((2, hbs, 2), lambda i: (0, i, 0)),
        pl.BlockSpec((nr, d), lambda i: (0, 0)),
        pl.BlockSpec((2 * d, d), lambda i: (0, 0)),
        pl.BlockSpec((d, d), lambda i: (0, 0)),
        pl.BlockSpec((d, 1), lambda i: (0, 0)),
        pl.BlockSpec((3 * d, d), lambda i: (0, 0)),
        pl.BlockSpec((8, d), lambda i: (0, 0)),
    ]
    if with_eu:
        in_specs.append(pl.BlockSpec((bb, d), lambda i: (i, 0)))
        out_specs = pl.BlockSpec((bb, 1), lambda i: (i, 0))
        out_shape = jax.ShapeDtypeStruct((b, 1), jnp.float32)
    else:
        out_specs = pl.BlockSpec((bb, d), lambda i: (i, 0))
        out_shape = jax.ShapeDtypeStruct((b, d), jnp.float32)
    return grid, in_specs, out_specs, out_shape


def kernel(u_entities, u_heads, u_relations, u_tails,
           v_entities, v_heads, v_relations, v_tails,
           entity_emb, rec_emb, rel_emb,
           W1, b1, W2, b2, W3, b3, Wagg, bagg):
    b, s = u_entities.shape
    d = entity_emb.shape[1]
    nr = rel_emb.shape[0]
    ls = u_heads.shape[0]
    n = b * s

    g_uh, g_ut = _sc_gather_pairs([
        (entity_emb, u_heads.reshape(-1, _W)),
        (entity_emb, u_tails.reshape(-1, _W)),
    ])
    (g_ue,) = _sc_gather_pairs([(rec_emb, u_entities.reshape(-1, _W))])
    g_vh, g_vt = _sc_gather_pairs([
        (entity_emb, v_heads.reshape(-1, _W)),
        (entity_emb, v_tails.reshape(-1, _W)),
    ])
    (g_ve,) = _sc_gather_pairs([(entity_emb, v_entities.reshape(-1, _W))])
    pk = lambda g: g.reshape(-1, 2 * d)

    urels = u_relations.astype(jnp.int8).reshape(ls, n // 2, 2)
    vrels = v_relations.astype(jnp.int8).reshape(ls, n // 2, 2)
    bmat = jnp.concatenate([
        b1[None, :], b2[None, :], bagg[None, :],
        jnp.broadcast_to(b3, (d,))[None, :], jnp.zeros((4, d), jnp.float32),
    ], axis=0)
    weights = (rel_emb, W1, W2, W3, Wagg, bmat)

    grid, in_specs_u, out_specs_u, out_shape_u = _tc_side_specs(b, s, d, nr, False)
    eu = pl.pallas_call(
        _tc_u_body, grid=grid, in_specs=in_specs_u,
        out_specs=out_specs_u, out_shape=out_shape_u,
    )(pk(g_uh), pk(g_uh), pk(g_ut), pk(g_ut), pk(g_ue), urels, *weights)

    bb = _BB
    hbs = bb * s // 2
    nlb = b * s // 2 // hbs
    evl = pl.pallas_call(
        _tc_vl_body, grid=(b // bb,),
        in_specs=[
            pl.BlockSpec((hbs, 2 * d), lambda i: (i, 0)),
            pl.BlockSpec((hbs, 2 * d), lambda i: (i + nlb, 0)),
            pl.BlockSpec((hbs, 2 * d), lambda i: (i, 0)),
            pl.BlockSpec((hbs, 2 * d), lambda i: (i + nlb, 0)),
            pl.BlockSpec((2, hbs, 2), lambda i: (0, i, 0)),
            pl.BlockSpec((nr, d), lambda i: (0, 0)),
            pl.BlockSpec((2 * d, d), lambda i: (0, 0)),
            pl.BlockSpec((d, d), lambda i: (0, 0)),
            pl.BlockSpec((d, 1), lambda i: (0, 0)),
            pl.BlockSpec((8, d), lambda i: (0, 0)),
        ],
        out_specs=pl.BlockSpec((bb, 2 * d), lambda i: (i, 0)),
        out_shape=jax.ShapeDtypeStruct((b, 2 * d), jnp.float32),
    )(pk(g_vh), pk(g_vh), pk(g_vt), pk(g_vt), vrels,
      rel_emb, W1, W2, W3, bmat)
    out = pl.pallas_call(
        _tc_vf_body, grid=(b // bb,),
        in_specs=[
            pl.BlockSpec((hbs, 2 * d), lambda i: (i, 0)),
            pl.BlockSpec((bb, 2 * d), lambda i: (i, 0)),
            pl.BlockSpec((bb, d), lambda i: (i, 0)),
            pl.BlockSpec((3 * d, d), lambda i: (0, 0)),
            pl.BlockSpec((8, d), lambda i: (0, 0)),
        ],
        out_specs=pl.BlockSpec((bb, 1), lambda i: (i, 0)),
        out_shape=jax.ShapeDtypeStruct((b, 1), jnp.float32),
    )(pk(g_ve), evl, eu, Wagg, bmat)
    return out.reshape(-1)
